# Initial kernel scaffold; baseline (speedup 1.0000x reference)
import jax
import jax.numpy as jnp
from jax.experimental import pallas as pl


def kernel(nfeats, efeats, edge_index, W_apply_w, W_apply_b, W_edge_w, W_edge_b):
    raise NotImplementedError("write your pallas kernel here")


# speedup vs baseline: 2.4827x; 2.4827x over previous
import functools

import jax
import jax.numpy as jnp
from jax import lax
from jax.experimental import pallas as pl
from jax.experimental.pallas import tpu as pltpu
from jax.experimental.pallas import tpu_sc as plsc

_N = 10000
_E = 320000
_DIN = 128
_DE = 16
_DOUT = 128
_DEDGE = 256

_NC = 2
_NS = 16
_NW = _NC * _NS
_EPW = _E // _NW
_CHUNK = 80
_NCHUNK = _EPW // _CHUNK
_NPAD = 10240
_RPT = _NPAD // _NS

_mesh = plsc.VectorSubcoreMesh(
    core_axis_name="c", subcore_axis_name="s", num_cores=_NC, num_subcores=_NS
)


@functools.partial(
    pl.kernel,
    out_type=[
        jax.ShapeDtypeStruct((_NC, _NPAD, _DE), jnp.float32),
        jax.ShapeDtypeStruct((_NC, _NPAD, _DE), jnp.float32),
    ],
    mesh=_mesh,
    scratch_types=[
        pltpu.VMEM((_CHUNK, _DE), jnp.float32),
        pltpu.VMEM((_CHUNK, _DE), jnp.float32),
        pltpu.VMEM((_CHUNK,), jnp.int32),
        pltpu.VMEM_SHARED((_NPAD, _DE), jnp.float32),
        pltpu.VMEM_SHARED((_NPAD, _DE), jnp.float32),
    ],
)
def _seg_sum(ef_hbm, vidx_hbm, z_hbm, sums_hbm, cnts_hbm,
             efbuf, onebuf, idxbuf, acc_f, acc_c):
    c = lax.axis_index("c")
    s = lax.axis_index("s")
    wid = s * _NC + c

    r0 = pl.multiple_of(s * _RPT, 8)
    pltpu.sync_copy(z_hbm.at[pl.ds(r0, _RPT)], acc_f.at[pl.ds(r0, _RPT)])
    pltpu.sync_copy(z_hbm.at[pl.ds(r0, _RPT)], acc_c.at[pl.ds(r0, _RPT)])

    lane = lax.broadcasted_iota(jnp.int32, (16,), 0)
    onerow = jnp.where(lane == 0, 1.0, 0.0).astype(jnp.float32)

    def fill_body(r, carry):
        onebuf[r, :] = onerow
        return carry

    lax.fori_loop(0, _CHUNK, fill_body, 0)
    plsc.subcore_barrier()

    def chunk_body(ci, carry):
        base = pl.multiple_of(wid * _EPW + ci * _CHUNK, 8)
        pltpu.sync_copy(vidx_hbm.at[pl.ds(base, _CHUNK)], idxbuf)
        pltpu.sync_copy(ef_hbm.at[pl.ds(base, _CHUNK)], efbuf)
        pltpu.sync_copy(efbuf, acc_f.at[idxbuf], add=True)
        pltpu.sync_copy(onebuf, acc_c.at[idxbuf], add=True)
        return carry

    lax.fori_loop(0, _NCHUNK, chunk_body, 0)
    plsc.subcore_barrier()

    pltpu.sync_copy(acc_f.at[pl.ds(r0, _RPT)], sums_hbm.at[c].at[pl.ds(r0, _RPT)])
    pltpu.sync_copy(acc_c.at[pl.ds(r0, _RPT)], cnts_hbm.at[c].at[pl.ds(r0, _RPT)])


_BN = 1000


def _apply_body(nf_ref, sums_ref, cnts_ref, w1_ref, w2_ref, ba_ref,
                wu_ref, wv_ref, be_ref, h_ref, a_ref, b_ref):
    sums = sums_ref[0] + sums_ref[1]
    cnt = cnts_ref[0, :, 0:1] + cnts_ref[1, :, 0:1]
    h_neigh = sums / jnp.maximum(cnt, 1.0)
    dn = (((1,), (1,)), ((), ()))
    acc = lax.dot_general(nf_ref[...], w1_ref[...], dn,
                          preferred_element_type=jnp.float32)
    acc += lax.dot_general(h_neigh, w2_ref[...], dn,
                           preferred_element_type=jnp.float32)
    h = jnp.maximum(acc + ba_ref[...], 0.0)
    h_ref[...] = h
    a_ref[...] = lax.dot_general(h, wu_ref[...], dn,
                                 preferred_element_type=jnp.float32) + be_ref[...]
    b_ref[...] = lax.dot_general(h, wv_ref[...], dn,
                                 preferred_element_type=jnp.float32)


_apply_call = pl.pallas_call(
    _apply_body,
    grid=(_N // _BN,),
    in_specs=[
        pl.BlockSpec((_BN, _DIN), lambda i: (i, 0)),
        pl.BlockSpec((_NC, _BN, _DE), lambda i: (0, i, 0)),
        pl.BlockSpec((_NC, _BN, _DE), lambda i: (0, i, 0)),
        pl.BlockSpec((_DOUT, _DIN), lambda i: (0, 0)),
        pl.BlockSpec((_DOUT, _DE), lambda i: (0, 0)),
        pl.BlockSpec((1, _DOUT), lambda i: (0, 0)),
        pl.BlockSpec((_DEDGE, _DOUT), lambda i: (0, 0)),
        pl.BlockSpec((_DEDGE, _DOUT), lambda i: (0, 0)),
        pl.BlockSpec((1, _DEDGE), lambda i: (0, 0)),
    ],
    out_specs=[
        pl.BlockSpec((_BN, _DOUT), lambda i: (i, 0)),
        pl.BlockSpec((_BN, _DEDGE), lambda i: (i, 0)),
        pl.BlockSpec((_BN, _DEDGE), lambda i: (i, 0)),
    ],
    out_shape=[
        jax.ShapeDtypeStruct((_N, _DOUT), jnp.float32),
        jax.ShapeDtypeStruct((_N, _DEDGE), jnp.float32),
        jax.ShapeDtypeStruct((_N, _DEDGE), jnp.float32),
    ],
)


@functools.partial(
    pl.kernel,
    out_type=jax.ShapeDtypeStruct((_E, _DEDGE), jnp.float32),
    mesh=_mesh,
    scratch_types=[
        pltpu.VMEM((_CHUNK,), jnp.int32),
        pltpu.VMEM((_CHUNK,), jnp.int32),
        pltpu.VMEM((_CHUNK, _DEDGE), jnp.float32),
        pltpu.VMEM((_CHUNK, _DEDGE), jnp.float32),
        pltpu.SemaphoreType.DMA,
        pltpu.SemaphoreType.DMA,
    ],
)
def _edge_mlp(a_hbm, b_hbm, u_hbm, v_hbm, out_hbm,
              idxu, idxv, bufa, bufb, sema, semb):
    c = lax.axis_index("c")
    s = lax.axis_index("s")
    wid = s * _NC + c

    def chunk_body(ci, carry):
        base = pl.multiple_of(wid * _EPW + ci * _CHUNK, 8)
        pltpu.sync_copy(u_hbm.at[pl.ds(base, _CHUNK)], idxu)
        pltpu.sync_copy(v_hbm.at[pl.ds(base, _CHUNK)], idxv)
        cpa = pltpu.async_copy(a_hbm.at[idxu], bufa, sema)
        cpb = pltpu.async_copy(b_hbm.at[idxv], bufb, semb)
        cpa.wait()
        cpb.wait()

        def row_body(r, carry2):
            for k in range(_DEDGE // 16):
                sl = pl.ds(k * 16, 16)
                bufa[r, sl] = bufa[r, sl] + bufb[r, sl]
            return carry2

        lax.fori_loop(0, _CHUNK, row_body, 0)
        pltpu.sync_copy(bufa, out_hbm.at[pl.ds(base, _CHUNK)])
        return carry

    lax.fori_loop(0, _NCHUNK, chunk_body, 0)


def kernel(nfeats, efeats, edge_index, W_apply_w, W_apply_b, W_edge_w, W_edge_b):
    nf = nfeats.reshape(_N, _DIN)
    ef = efeats.reshape(_E, _DE)
    u = edge_index[0]
    v = edge_index[1]
    z = jnp.zeros((_NPAD, _DE), jnp.float32)

    sums, cnts = _seg_sum(ef, v, z)
    h, a, b = _apply_call(
        nf, sums, cnts,
        W_apply_w[:, :_DIN], W_apply_w[:, _DIN:],
        W_apply_b.reshape(1, _DOUT),
        W_edge_w[:, :_DOUT], W_edge_w[:, _DOUT:],
        W_edge_b.reshape(1, _DEDGE),
    )
    edge = _edge_mlp(a, b, u, v)
    return h.reshape(_N, 1, _DOUT), edge.reshape(_E, 1, _DEDGE)
